# Initial kernel scaffold; baseline (speedup 1.0000x reference)
import jax
import jax.numpy as jnp
from jax.experimental import pallas as pl


def kernel(pred_boxes, pred_scores, pred_labels, tgt_boxes, tgt_labels):
    raise NotImplementedError("write your pallas kernel here")


# speedup vs baseline: 5.8645x; 5.8645x over previous
import functools

import jax
import jax.numpy as jnp
from jax.experimental import pallas as pl
from jax.experimental.pallas import tpu as pltpu

_CHUNK = 200
_GRID = 10
_M = 500
_MPAD = 512
_TPAD = 16
_RPAD = 128


def _body(pb_ref, plab_ref, tbt_ref, tlab_ref, thr_ref, rec_ref, out_ref,
          iou_s, matched_s, ctp_s, acc_s):
    g = pl.program_id(0)

    @pl.when(g == 0)
    def _init():
        matched_s[...] = jnp.zeros_like(matched_s)
        ctp_s[...] = jnp.zeros_like(ctp_s)
        acc_s[...] = jnp.zeros_like(acc_s)

    for c in range(3):
        px0 = pb_ref[c, :, 0:1]
        py0 = pb_ref[c, :, 1:2]
        px1 = pb_ref[c, :, 2:3]
        py1 = pb_ref[c, :, 3:4]
        tx0 = tbt_ref[c, 0:1, :]
        ty0 = tbt_ref[c, 1:2, :]
        tx1 = tbt_ref[c, 2:3, :]
        ty1 = tbt_ref[c, 3:4, :]
        area_p = (px1 - px0) * (py1 - py0)
        area_t = (tx1 - tx0) * (ty1 - ty0)
        w = jnp.clip(jnp.minimum(px1, tx1) - jnp.maximum(px0, tx0), 0.0)
        h = jnp.clip(jnp.minimum(py1, ty1) - jnp.maximum(py0, ty0), 0.0)
        inter = w * h
        union = area_p + area_t - inter
        iou = inter / jnp.maximum(union, 1e-9)
        lm = plab_ref[c, :, 0:1] == tlab_ref[c, 0:1, :]
        iou_m = jnp.where(lm, iou, -1.0)
        iou_s[:, c:c + 1, :] = iou_m[:, None, :]

    thr = thr_ref[...]
    rec = rec_ref[...]
    lane = jax.lax.broadcasted_iota(jnp.int32, (1, 1, _MPAD), 2)

    def step(j, _):
        row = iou_s[j]
        iou_row = row[0:3, :].reshape(3, 1, _MPAD)
        matched = matched_s[...]
        avail = jnp.logical_and(iou_row >= thr, matched < 0.5)
        masked = jnp.where(avail, jnp.broadcast_to(iou_row, matched.shape),
                           -1.0)
        bestv = jnp.max(masked, axis=2, keepdims=True)
        has = bestv > 0.0
        idxs = jnp.where(masked == bestv, jnp.broadcast_to(lane, masked.shape),
                         jnp.int32(_MPAD * 2))
        besti = jnp.min(idxs, axis=2, keepdims=True)
        newm = jnp.logical_and(lane == besti, has)
        matched_s[...] = jnp.where(newm, 1.0, matched)
        ctp = ctp_s[...] + has.astype(jnp.float32)
        ctp_s[...] = ctp
        denom = (g * _CHUNK + j + 1).astype(jnp.float32)
        prec = ctp / denom
        recall = ctp / float(_M)
        contrib = jnp.where(recall >= rec, prec, 0.0)
        acc_s[...] = jnp.maximum(acc_s[...], contrib)
        return 0

    jax.lax.fori_loop(0, _CHUNK, step, 0)

    @pl.when(g == _GRID - 1)
    def _fin():
        ap = jnp.sum(acc_s[:, 0:10, 0:101], axis=(1, 2)) / float(10 * 101)
        out_ref[...] = jnp.broadcast_to(ap[:, None], (3, 128))


@jax.jit
def kernel(pred_boxes, pred_scores, pred_labels, tgt_boxes, tgt_labels):
    order = jnp.argsort(-pred_scores, axis=1)
    pb = jnp.take_along_axis(pred_boxes, order[..., None], axis=1)
    plab = jnp.take_along_axis(pred_labels, order, axis=1)[..., None]
    tbt = jnp.pad(jnp.transpose(tgt_boxes, (0, 2, 1)),
                  ((0, 0), (0, 0), (0, _MPAD - _M)))
    tlab = jnp.pad(tgt_labels, ((0, 0), (0, _MPAD - _M)),
                   constant_values=-1)[:, None, :]
    thr = jnp.pad(jnp.arange(0.5, 0.999, 0.05, dtype=jnp.float32),
                  (0, _TPAD - 10), constant_values=2.0).reshape(1, _TPAD, 1)
    rec = jnp.pad(jnp.linspace(0.0, 1.0, 101, dtype=jnp.float32),
                  (0, _RPAD - 101), constant_values=2.0).reshape(1, 1, _RPAD)

    out = pl.pallas_call(
        _body,
        grid=(_GRID,),
        in_specs=[
            pl.BlockSpec((3, _CHUNK, 4), lambda i: (0, i, 0)),
            pl.BlockSpec((3, _CHUNK, 1), lambda i: (0, i, 0)),
            pl.BlockSpec((3, 4, _MPAD), lambda i: (0, 0, 0)),
            pl.BlockSpec((3, 1, _MPAD), lambda i: (0, 0, 0)),
            pl.BlockSpec((1, _TPAD, 1), lambda i: (0, 0, 0)),
            pl.BlockSpec((1, 1, _RPAD), lambda i: (0, 0, 0)),
        ],
        out_specs=pl.BlockSpec((3, 128), lambda i: (0, 0)),
        out_shape=jax.ShapeDtypeStruct((3, 128), jnp.float32),
        scratch_shapes=[
            pltpu.VMEM((_CHUNK, 8, _MPAD), jnp.float32),
            pltpu.VMEM((3, _TPAD, _MPAD), jnp.float32),
            pltpu.VMEM((3, _TPAD, 1), jnp.float32),
            pltpu.VMEM((3, _TPAD, _RPAD), jnp.float32),
        ],
    )(pb, plab, tbt, tlab, thr, rec)
    return out[:, 0]


# speedup vs baseline: 38.4368x; 6.5541x over previous
import jax
import jax.numpy as jnp
from jax.experimental import pallas as pl
from jax.experimental.pallas import tpu as pltpu

_N = 2000
_CHUNK = 200
_GRID = 10
_M = 500
_MPAD = 512
_TPAD = 16
_RPAD = 128


def _flags_body(pb_ref, plab_ref, tbt_ref, tlab_ref, flag_ref):
    acc = jnp.zeros((_CHUNK, _MPAD), dtype=jnp.bool_)
    for c in range(3):
        px0 = pb_ref[c, :, 0:1]
        py0 = pb_ref[c, :, 1:2]
        px1 = pb_ref[c, :, 2:3]
        py1 = pb_ref[c, :, 3:4]
        tx0 = tbt_ref[c, 0:1, :]
        ty0 = tbt_ref[c, 1:2, :]
        tx1 = tbt_ref[c, 2:3, :]
        ty1 = tbt_ref[c, 3:4, :]
        area_p = (px1 - px0) * (py1 - py0)
        area_t = (tx1 - tx0) * (ty1 - ty0)
        w = jnp.minimum(px1, tx1) - jnp.maximum(px0, tx0)
        h = jnp.minimum(py1, ty1) - jnp.maximum(py0, ty0)
        inter = w * h
        asum = area_p + area_t
        lm = plab_ref[c, :, 0:1] == tlab_ref[c, 0:1, :]
        valid = (w > 0.0) & (h > 0.0) & (inter * 3.0 >= asum * (1.0 - 1e-5)) & lm
        acc = jnp.logical_or(acc, valid)
    flag_ref[...] = jnp.any(acc, axis=1, keepdims=True).astype(jnp.int32)


def _scan_body(flag_ref, coords_ref, tbt_ref, tlabf_ref, thr_ref, rec_ref,
               out_ref, pos_s, matched_s, ctp_s, acc_s):
    matched_s[...] = jnp.zeros_like(matched_s)
    ctp_s[...] = jnp.zeros_like(ctp_s)
    acc_s[...] = jnp.zeros_like(acc_s)

    def comp(i, k):
        def app(kk):
            pos_s[kk] = i
            return kk + 1
        return jax.lax.cond(flag_ref[i] > 0, app, lambda kk: kk, k)

    num = jax.lax.fori_loop(0, _N, comp, 0)

    thr = thr_ref[...]
    rec = rec_ref[...]
    lane = jax.lax.broadcasted_iota(jnp.int32, (1, 1, _MPAD), 2)
    tx0 = tbt_ref[:, 0:1, :]
    ty0 = tbt_ref[:, 1:2, :]
    tx1 = tbt_ref[:, 2:3, :]
    ty1 = tbt_ref[:, 3:4, :]
    area_t = (tx1 - tx0) * (ty1 - ty0)
    tl = tlabf_ref[...]

    def step(k, _):
        p = pos_s[k]
        tile = coords_ref[p]
        px0 = tile[0:3, 0:1].reshape(3, 1, 1)
        py0 = tile[0:3, 1:2].reshape(3, 1, 1)
        px1 = tile[0:3, 2:3].reshape(3, 1, 1)
        py1 = tile[0:3, 3:4].reshape(3, 1, 1)
        lab = tile[0:3, 4:5].reshape(3, 1, 1)
        area_p = (px1 - px0) * (py1 - py0)
        w = jnp.clip(jnp.minimum(px1, tx1) - jnp.maximum(px0, tx0), 0.0)
        h = jnp.clip(jnp.minimum(py1, ty1) - jnp.maximum(py0, ty0), 0.0)
        inter = w * h
        union = area_p + area_t - inter
        iou = inter / jnp.maximum(union, 1e-9)
        iou_m = jnp.where(lab == tl, iou, -1.0)
        matched = matched_s[...]
        avail = jnp.logical_and(iou_m >= thr, matched < 0.5)
        masked = jnp.where(avail, jnp.broadcast_to(iou_m, matched.shape), -1.0)
        bestv = jnp.max(masked, axis=2, keepdims=True)
        has = bestv > 0.0
        idxs = jnp.where(masked == bestv,
                         jnp.broadcast_to(lane, masked.shape),
                         jnp.int32(_MPAD * 2))
        besti = jnp.min(idxs, axis=2, keepdims=True)
        newm = jnp.logical_and(lane == besti, has)
        matched_s[...] = jnp.where(newm, 1.0, matched)
        ctp = ctp_s[...] + has.astype(jnp.float32)
        ctp_s[...] = ctp
        denom = (p + 1).astype(jnp.float32)
        prec = ctp / denom
        recall = ctp / float(_M)
        acc_s[...] = jnp.maximum(acc_s[...],
                                 jnp.where(recall >= rec, prec, 0.0))
        return 0

    jax.lax.fori_loop(0, num, step, 0)

    ap = jnp.sum(acc_s[:, 0:10, 0:101], axis=(1, 2)) / float(10 * 101)
    out_ref[...] = jnp.broadcast_to(ap[:, None], (3, 128))


@jax.jit
def kernel(pred_boxes, pred_scores, pred_labels, tgt_boxes, tgt_labels):
    order = jnp.argsort(-pred_scores, axis=1)
    pb = jnp.take_along_axis(pred_boxes, order[..., None], axis=1)
    plab = jnp.take_along_axis(pred_labels, order, axis=1)[..., None]
    tbt = jnp.pad(jnp.transpose(tgt_boxes, (0, 2, 1)),
                  ((0, 0), (0, 0), (0, _MPAD - _M)))
    tlab = jnp.pad(tgt_labels, ((0, 0), (0, _MPAD - _M)),
                   constant_values=-1)[:, None, :]
    thr = jnp.pad(jnp.arange(0.5, 0.999, 0.05, dtype=jnp.float32),
                  (0, _TPAD - 10), constant_values=2.0).reshape(1, _TPAD, 1)
    rec = jnp.pad(jnp.linspace(0.0, 1.0, 101, dtype=jnp.float32),
                  (0, _RPAD - 101), constant_values=2.0).reshape(1, 1, _RPAD)

    flags = pl.pallas_call(
        _flags_body,
        grid=(_GRID,),
        in_specs=[
            pl.BlockSpec((3, _CHUNK, 4), lambda i: (0, i, 0)),
            pl.BlockSpec((3, _CHUNK, 1), lambda i: (0, i, 0)),
            pl.BlockSpec((3, 4, _MPAD), lambda i: (0, 0, 0)),
            pl.BlockSpec((3, 1, _MPAD), lambda i: (0, 0, 0)),
        ],
        out_specs=pl.BlockSpec((_CHUNK, 1), lambda i: (i, 0)),
        out_shape=jax.ShapeDtypeStruct((_N, 1), jnp.int32),
    )(pb, plab, tbt, tlab)

    coords = jnp.concatenate([pb, plab.astype(jnp.float32)], axis=-1)
    coords = jnp.transpose(coords, (1, 0, 2))
    coords = jnp.pad(coords, ((0, 0), (0, 5), (0, 123)))

    out = pl.pallas_call(
        _scan_body,
        in_specs=[
            pl.BlockSpec(memory_space=pltpu.SMEM),
            pl.BlockSpec((_N, 8, 128), lambda: (0, 0, 0)),
            pl.BlockSpec((3, 4, _MPAD), lambda: (0, 0, 0)),
            pl.BlockSpec((3, 1, _MPAD), lambda: (0, 0, 0)),
            pl.BlockSpec((1, _TPAD, 1), lambda: (0, 0, 0)),
            pl.BlockSpec((1, 1, _RPAD), lambda: (0, 0, 0)),
        ],
        out_specs=pl.BlockSpec((3, 128), lambda: (0, 0)),
        out_shape=jax.ShapeDtypeStruct((3, 128), jnp.float32),
        scratch_shapes=[
            pltpu.SMEM((_N + 48,), jnp.int32),
            pltpu.VMEM((3, _TPAD, _MPAD), jnp.float32),
            pltpu.VMEM((3, _TPAD, 1), jnp.float32),
            pltpu.VMEM((3, _TPAD, _RPAD), jnp.float32),
        ],
    )(flags[:, 0], coords, tbt, tlab.astype(jnp.float32), thr, rec)
    return out[:, 0]


# speedup vs baseline: 85.3642x; 2.2209x over previous
import jax
import jax.numpy as jnp
from jax.experimental import pallas as pl
from jax.experimental.pallas import tpu as pltpu

_N = 2000
_NPAD = 2048
_M = 500
_MPAD = 512
_TPAD = 16
_RPAD = 128
_BIG = 1 << 28


def _body(cmb_ref, pbT_ref, plT_ref, tbS_ref, tlS_ref, tbt_ref, tlabf_ref,
          sv_ref, thr_ref, rec_ref, out_ref, matched_s, ctp_s, acc_s):
    matched_s[...] = jnp.zeros_like(matched_s)
    ctp_s[...] = jnp.zeros_like(ctp_s)
    acc_s[...] = jnp.zeros_like(acc_s)

    flat = (jax.lax.broadcasted_iota(jnp.int32, (_TPAD, _RPAD), 0) * _RPAD
            + jax.lax.broadcasted_iota(jnp.int32, (_TPAD, _RPAD), 1))

    acts = []
    for c in range(3):
        tx0 = tbS_ref[c, :, 0:1]
        ty0 = tbS_ref[c, :, 1:2]
        tx1 = tbS_ref[c, :, 2:3]
        ty1 = tbS_ref[c, :, 3:4]
        atc = ((tx1 - tx0) * (ty1 - ty0)) * (1.0 - 1e-5)
        tl = tlS_ref[c, :, 0:1]
        rows = []
        for g in range(_TPAD):
            s = slice(_RPAD * g, _RPAD * (g + 1))
            px0 = pbT_ref[c, 0:1, s]
            py0 = pbT_ref[c, 1:2, s]
            px1 = pbT_ref[c, 2:3, s]
            py1 = pbT_ref[c, 3:4, s]
            apc = ((px1 - px0) * (py1 - py0)) * (1.0 - 1e-5)
            w = jnp.maximum(jnp.minimum(px1, tx1) - jnp.maximum(px0, tx0), 0.0)
            h = jnp.maximum(jnp.minimum(py1, ty1) - jnp.maximum(py0, ty0), 0.0)
            inter = w * h
            lm = plT_ref[c, 0:1, s] == tl
            valid = jnp.logical_and(inter * 3.0 >= apc + atc, lm)
            rows.append(jnp.any(valid, axis=0, keepdims=True
                                ).astype(jnp.float32))
        acts.append(jnp.concatenate(rows, axis=0))

    kc_vec = [jnp.sum(a, keepdims=True).astype(jnp.int32).reshape(1, 1)
              for a in acts]
    kc = [jnp.sum(a).astype(jnp.int32) for a in acts]
    kmax = jnp.maximum(kc[0], jnp.maximum(kc[1], kc[2]))

    thr = thr_ref[...]
    rec = rec_ref[...]
    lane = jax.lax.broadcasted_iota(jnp.int32, (1, 1, _MPAD), 2)
    tx0s = tbt_ref[:, 0:1, :]
    ty0s = tbt_ref[:, 1:2, :]
    tx1s = tbt_ref[:, 2:3, :]
    ty1s = tbt_ref[:, 3:4, :]
    area_t = (tx1s - tx0s) * (ty1s - ty0s)
    tls = tlabf_ref[...]

    def step(k, carry):
        new_carry = []
        cols = {q: [] for q in range(6)}
        ranks = []
        valids = []
        for c in range(3):
            act = carry[c] > 0.5
            svc = sv_ref[c]
            sel = jnp.where(act, svc, -1.0)
            smax = jnp.max(sel)
            cand = jnp.logical_and(sel == smax, act)
            p_raw = jnp.min(jnp.where(cand, flat, _BIG))
            p = jnp.minimum(p_raw, _N - 1)
            new_carry.append(jnp.where(flat == p_raw, 0.0, carry[c]))
            row = cmb_ref[c, pl.ds(p, 1), :]
            for q in range(6):
                cols[q].append(row[0:1, q:q + 1])
            s_c = row[0:1, 5:6]
            gt = jnp.sum((svc > s_c).astype(jnp.int32), keepdims=True
                         ).reshape(1, 1)
            tie = jnp.sum(jnp.logical_and(svc == s_c, flat < p
                                          ).astype(jnp.int32), keepdims=True
                          ).reshape(1, 1)
            ranks.append(gt + tie)
            valids.append((jnp.full((1, 1), k, jnp.int32) < kc_vec[c]
                           ).astype(jnp.float32))
        px0 = jnp.concatenate(cols[0], axis=0).reshape(3, 1, 1)
        py0 = jnp.concatenate(cols[1], axis=0).reshape(3, 1, 1)
        px1 = jnp.concatenate(cols[2], axis=0).reshape(3, 1, 1)
        py1 = jnp.concatenate(cols[3], axis=0).reshape(3, 1, 1)
        lab = jnp.concatenate(cols[4], axis=0).reshape(3, 1, 1)
        rank3 = jnp.concatenate(ranks, axis=0).reshape(3, 1, 1)
        valid3 = jnp.concatenate(valids, axis=0).reshape(3, 1, 1) > 0.5

        area_p = (px1 - px0) * (py1 - py0)
        w = jnp.clip(jnp.minimum(px1, tx1s) - jnp.maximum(px0, tx0s), 0.0)
        h = jnp.clip(jnp.minimum(py1, ty1s) - jnp.maximum(py0, ty0s), 0.0)
        inter = w * h
        union = area_p + area_t - inter
        iou = inter / jnp.maximum(union, 1e-9)
        iou_m = jnp.where(lab == tls, iou, -1.0)
        iou_m = jnp.where(valid3, iou_m, -1.0)

        matched = matched_s[...]
        avail = jnp.logical_and(iou_m >= thr, matched < 0.5)
        masked = jnp.where(avail, jnp.broadcast_to(iou_m, matched.shape), -1.0)
        bestv = jnp.max(masked, axis=2, keepdims=True)
        has = bestv > 0.0
        idxs = jnp.where(masked == bestv,
                         jnp.broadcast_to(lane, masked.shape),
                         jnp.int32(_MPAD * 2))
        besti = jnp.min(idxs, axis=2, keepdims=True)
        newm = jnp.logical_and(lane == besti, has)
        matched_s[...] = jnp.where(newm, 1.0, matched)
        ctp = ctp_s[...] + has.astype(jnp.float32)
        ctp_s[...] = ctp
        denom = jnp.where(valid3, (rank3 + 1).astype(jnp.float32), 4096.0)
        prec = ctp / denom
        recall = ctp / float(_M)
        acc_s[...] = jnp.maximum(acc_s[...],
                                 jnp.where(recall >= rec, prec, 0.0))
        return tuple(new_carry)

    jax.lax.fori_loop(0, kmax, step, (acts[0], acts[1], acts[2]))

    ap = jnp.sum(acc_s[:, 0:10, 0:101], axis=(1, 2)) / float(10 * 101)
    out_ref[...] = jnp.broadcast_to(ap[:, None], (3, 128))


@jax.jit
def kernel(pred_boxes, pred_scores, pred_labels, tgt_boxes, tgt_labels):
    f32 = jnp.float32
    cmb = jnp.concatenate(
        [pred_boxes, pred_labels[..., None].astype(f32),
         pred_scores[..., None]], axis=-1)
    pbT = jnp.pad(jnp.transpose(pred_boxes, (0, 2, 1)),
                  ((0, 0), (0, 0), (0, _NPAD - _N)),
                  constant_values=-1e6)
    plT = jnp.pad(pred_labels.astype(f32)[:, None, :],
                  ((0, 0), (0, 0), (0, _NPAD - _N)),
                  constant_values=-2.0)
    tbS = jnp.pad(tgt_boxes, ((0, 0), (0, _MPAD - _M), (0, 0)))
    tlS = jnp.pad(tgt_labels, ((0, 0), (0, _MPAD - _M)),
                  constant_values=-1).astype(f32)[..., None]
    tbt = jnp.pad(jnp.transpose(tgt_boxes, (0, 2, 1)),
                  ((0, 0), (0, 0), (0, _MPAD - _M)))
    tlabf = jnp.pad(tgt_labels, ((0, 0), (0, _MPAD - _M)),
                    constant_values=-1).astype(f32)[:, None, :]
    sv = jnp.pad(pred_scores, ((0, 0), (0, _NPAD - _N)),
                 constant_values=-1.0).reshape(3, _TPAD, _RPAD)
    thr = jnp.pad(jnp.arange(0.5, 0.999, 0.05, dtype=f32),
                  (0, _TPAD - 10), constant_values=2.0).reshape(1, _TPAD, 1)
    rec = jnp.pad(jnp.linspace(0.0, 1.0, 101, dtype=f32),
                  (0, _RPAD - 101), constant_values=2.0).reshape(1, 1, _RPAD)

    full = lambda shape: pl.BlockSpec(shape, lambda: (0,) * len(shape))
    out = pl.pallas_call(
        _body,
        in_specs=[
            full((3, _N, 6)),
            full((3, 4, _NPAD)),
            full((3, 1, _NPAD)),
            full((3, _MPAD, 4)),
            full((3, _MPAD, 1)),
            full((3, 4, _MPAD)),
            full((3, 1, _MPAD)),
            full((3, _TPAD, _RPAD)),
            full((1, _TPAD, 1)),
            full((1, 1, _RPAD)),
        ],
        out_specs=pl.BlockSpec((3, 128), lambda: (0, 0)),
        out_shape=jax.ShapeDtypeStruct((3, 128), jnp.float32),
        scratch_shapes=[
            pltpu.VMEM((3, _TPAD, _MPAD), jnp.float32),
            pltpu.VMEM((3, _TPAD, 1), jnp.float32),
            pltpu.VMEM((3, _TPAD, _RPAD), jnp.float32),
        ],
    )(cmb, pbT, plT, tbS, tlS, tbt, tlabf, sv, thr, rec)
    return out[:, 0]


# speedup vs baseline: 98.7506x; 1.1568x over previous
import jax
import jax.numpy as jnp
from jax.experimental import pallas as pl
from jax.experimental.pallas import tpu as pltpu

_N = 2000
_NPAD = 2048
_M = 500
_MPAD = 512
_TPAD = 16
_RPAD = 128
_BIG = 1 << 28


def _body(pb_ref, labf_ref, pbT_ref, plT_ref, tbt_ref, tlabf_ref,
          sv_ref, thr_ref, rec_ref, out_ref, matched_s, ctp_s, acc_s):
    matched_s[...] = jnp.zeros_like(matched_s)
    ctp_s[...] = jnp.zeros_like(ctp_s)
    acc_s[...] = jnp.zeros_like(acc_s)

    flat = (jax.lax.broadcasted_iota(jnp.int32, (_TPAD, _RPAD), 0) * _RPAD
            + jax.lax.broadcasted_iota(jnp.int32, (_TPAD, _RPAD), 1))

    acts = []
    for c in range(3):
        tx0 = jnp.transpose(tbt_ref[c, 0:1, :], (1, 0))
        ty0 = jnp.transpose(tbt_ref[c, 1:2, :], (1, 0))
        tx1 = jnp.transpose(tbt_ref[c, 2:3, :], (1, 0))
        ty1 = jnp.transpose(tbt_ref[c, 3:4, :], (1, 0))
        atc = ((tx1 - tx0) * (ty1 - ty0)) * (1.0 - 1e-5)
        tl = jnp.transpose(tlabf_ref[c], (1, 0))
        rows = []
        for g in range(_TPAD):
            s = slice(_RPAD * g, _RPAD * (g + 1))
            px0 = pbT_ref[c, 0:1, s]
            py0 = pbT_ref[c, 1:2, s]
            px1 = pbT_ref[c, 2:3, s]
            py1 = pbT_ref[c, 3:4, s]
            apc = ((px1 - px0) * (py1 - py0)) * (1.0 - 1e-5)
            w = jnp.maximum(jnp.minimum(px1, tx1) - jnp.maximum(px0, tx0), 0.0)
            h = jnp.maximum(jnp.minimum(py1, ty1) - jnp.maximum(py0, ty0), 0.0)
            inter = w * h
            lm = plT_ref[c, 0:1, s] == tl
            valid = jnp.logical_and(inter * 3.0 >= apc + atc, lm)
            rows.append(jnp.any(valid, axis=0, keepdims=True
                                ).astype(jnp.float32))
        acts.append(jnp.concatenate(rows, axis=0))

    kc_vec = [jnp.sum(a, keepdims=True).astype(jnp.int32).reshape(1, 1)
              for a in acts]
    kc = [jnp.sum(a).astype(jnp.int32) for a in acts]
    kmax = jnp.maximum(kc[0], jnp.maximum(kc[1], kc[2]))

    thr = thr_ref[...]
    rec = rec_ref[...]
    lane = jax.lax.broadcasted_iota(jnp.int32, (1, 1, _MPAD), 2)
    tx0s = tbt_ref[:, 0:1, :]
    ty0s = tbt_ref[:, 1:2, :]
    tx1s = tbt_ref[:, 2:3, :]
    ty1s = tbt_ref[:, 3:4, :]
    area_t = (tx1s - tx0s) * (ty1s - ty0s)
    tls = tlabf_ref[...]

    def step(k, carry):
        new_carry = []
        cols = {q: [] for q in range(6)}
        ranks = []
        valids = []
        for c in range(3):
            act = carry[c] > 0.5
            svc = sv_ref[c]
            sel = jnp.where(act, svc, -1.0)
            smax = jnp.max(sel)
            cand = jnp.logical_and(sel == smax, act)
            p_raw = jnp.min(jnp.where(cand, flat, _BIG))
            p = jnp.minimum(p_raw, _N - 1)
            new_carry.append(jnp.where(flat == p_raw, 0.0, carry[c]))
            row = pb_ref[c, pl.ds(p, 1), :]
            for q in range(4):
                cols[q].append(row[0:1, q:q + 1])
            cols[4].append(labf_ref[c, pl.ds(p, 1), 0:1])
            s_c = jnp.max(sel, axis=(0, 1), keepdims=True)
            gt = jnp.sum((svc > s_c).astype(jnp.int32), keepdims=True
                         ).reshape(1, 1)
            tie = jnp.sum(jnp.logical_and(svc == s_c, flat < p
                                          ).astype(jnp.int32), keepdims=True
                          ).reshape(1, 1)
            ranks.append(gt + tie)
            valids.append((jnp.full((1, 1), k, jnp.int32) < kc_vec[c]
                           ).astype(jnp.float32))
        px0 = jnp.concatenate(cols[0], axis=0).reshape(3, 1, 1)
        py0 = jnp.concatenate(cols[1], axis=0).reshape(3, 1, 1)
        px1 = jnp.concatenate(cols[2], axis=0).reshape(3, 1, 1)
        py1 = jnp.concatenate(cols[3], axis=0).reshape(3, 1, 1)
        lab = jnp.concatenate(cols[4], axis=0).reshape(3, 1, 1)
        rank3 = jnp.concatenate(ranks, axis=0).reshape(3, 1, 1)
        valid3 = jnp.concatenate(valids, axis=0).reshape(3, 1, 1) > 0.5

        area_p = (px1 - px0) * (py1 - py0)
        w = jnp.clip(jnp.minimum(px1, tx1s) - jnp.maximum(px0, tx0s), 0.0)
        h = jnp.clip(jnp.minimum(py1, ty1s) - jnp.maximum(py0, ty0s), 0.0)
        inter = w * h
        union = area_p + area_t - inter
        iou = inter / jnp.maximum(union, 1e-9)
        iou_m = jnp.where(lab == tls, iou, -1.0)
        iou_m = jnp.where(valid3, iou_m, -1.0)

        matched = matched_s[...]
        avail = jnp.logical_and(iou_m >= thr, matched < 0.5)
        masked = jnp.where(avail, jnp.broadcast_to(iou_m, matched.shape), -1.0)
        bestv = jnp.max(masked, axis=2, keepdims=True)
        has = bestv > 0.0
        idxs = jnp.where(masked == bestv,
                         jnp.broadcast_to(lane, masked.shape),
                         jnp.int32(_MPAD * 2))
        besti = jnp.min(idxs, axis=2, keepdims=True)
        newm = jnp.logical_and(lane == besti, has)
        matched_s[...] = jnp.where(newm, 1.0, matched)
        ctp = ctp_s[...] + has.astype(jnp.float32)
        ctp_s[...] = ctp
        denom = jnp.where(valid3, (rank3 + 1).astype(jnp.float32), 4096.0)
        prec = ctp / denom
        recall = ctp / float(_M)
        acc_s[...] = jnp.maximum(acc_s[...],
                                 jnp.where(recall >= rec, prec, 0.0))
        return tuple(new_carry)

    jax.lax.fori_loop(0, kmax, step, (acts[0], acts[1], acts[2]))

    ap = jnp.sum(acc_s[:, 0:10, 0:101], axis=(1, 2)) / float(10 * 101)
    out_ref[...] = jnp.broadcast_to(ap[:, None], (3, 128))


@jax.jit
def kernel(pred_boxes, pred_scores, pred_labels, tgt_boxes, tgt_labels):
    f32 = jnp.float32
    labf = pred_labels.astype(f32)[..., None]
    pbT = jnp.pad(jnp.transpose(pred_boxes, (0, 2, 1)),
                  ((0, 0), (0, 0), (0, _NPAD - _N)),
                  constant_values=-1e6)
    plT = jnp.pad(pred_labels.astype(f32)[:, None, :],
                  ((0, 0), (0, 0), (0, _NPAD - _N)),
                  constant_values=-2.0)
    tbt = jnp.pad(jnp.transpose(tgt_boxes, (0, 2, 1)),
                  ((0, 0), (0, 0), (0, _MPAD - _M)))
    tlabf = jnp.pad(tgt_labels, ((0, 0), (0, _MPAD - _M)),
                    constant_values=-1).astype(f32)[:, None, :]
    sv = jnp.pad(pred_scores, ((0, 0), (0, _NPAD - _N)),
                 constant_values=-1.0).reshape(3, _TPAD, _RPAD)
    thr = jnp.pad(jnp.arange(0.5, 0.999, 0.05, dtype=f32),
                  (0, _TPAD - 10), constant_values=2.0).reshape(1, _TPAD, 1)
    rec = jnp.pad(jnp.linspace(0.0, 1.0, 101, dtype=f32),
                  (0, _RPAD - 101), constant_values=2.0).reshape(1, 1, _RPAD)

    full = lambda shape: pl.BlockSpec(shape, lambda: (0,) * len(shape))
    out = pl.pallas_call(
        _body,
        in_specs=[
            full((3, _N, 4)),
            full((3, _N, 1)),
            full((3, 4, _NPAD)),
            full((3, 1, _NPAD)),
            full((3, 4, _MPAD)),
            full((3, 1, _MPAD)),
            full((3, _TPAD, _RPAD)),
            full((1, _TPAD, 1)),
            full((1, 1, _RPAD)),
        ],
        out_specs=pl.BlockSpec((3, 128), lambda: (0, 0)),
        out_shape=jax.ShapeDtypeStruct((3, 128), jnp.float32),
        scratch_shapes=[
            pltpu.VMEM((3, _TPAD, _MPAD), jnp.float32),
            pltpu.VMEM((3, _TPAD, 1), jnp.float32),
            pltpu.VMEM((3, _TPAD, _RPAD), jnp.float32),
        ],
    )(pred_boxes, labf, pbT, plT, tbt, tlabf, sv, thr, rec)
    return out[:, 0]


# speedup vs baseline: 104.0925x; 1.0541x over previous
import jax
import jax.numpy as jnp
from jax.experimental import pallas as pl
from jax.experimental.pallas import tpu as pltpu

_N = 2000
_NPAD = 2048
_M = 500
_MPAD = 512
_TPAD = 16
_RPAD = 128
_BIG = 1 << 28


def _body(pb_ref, labf_ref, pbT_ref, plT_ref, tbt_ref, tlabf_ref,
          sv_ref, thr_ref, rec_ref, out_ref, matched_s, ctp_s, acc_s):
    matched_s[...] = jnp.zeros_like(matched_s)
    ctp_s[...] = jnp.zeros_like(ctp_s)
    acc_s[...] = jnp.zeros_like(acc_s)

    flat = (jax.lax.broadcasted_iota(jnp.int32, (_TPAD, _RPAD), 0) * _RPAD
            + jax.lax.broadcasted_iota(jnp.int32, (_TPAD, _RPAD), 1))

    acts = []
    for c in range(3):
        rows = [None] * _TPAD
        for t in range(_MPAD // _RPAD):
            st = slice(_RPAD * t, _RPAD * (t + 1))
            tx0 = jnp.transpose(tbt_ref[c, 0:1, st], (1, 0))
            ty0 = jnp.transpose(tbt_ref[c, 1:2, st], (1, 0))
            tx1 = jnp.transpose(tbt_ref[c, 2:3, st], (1, 0))
            ty1 = jnp.transpose(tbt_ref[c, 3:4, st], (1, 0))
            atc = ((tx1 - tx0) * (ty1 - ty0)) * (1.0 - 1e-5)
            tx0_3 = tx0 * 3.0
            tx1_3 = tx1 * 3.0
            tl = jnp.transpose(tlabf_ref[c, 0:1, st], (1, 0))
            for g in range(_TPAD):
                s = slice(_RPAD * g, _RPAD * (g + 1))
                px0 = pbT_ref[c, 0:1, s]
                py0 = pbT_ref[c, 1:2, s]
                px1 = pbT_ref[c, 2:3, s]
                py1 = pbT_ref[c, 3:4, s]
                apc = ((px1 - px0) * (py1 - py0)) * (1.0 - 1e-5)
                w3 = jnp.maximum(jnp.minimum(px1 * 3.0, tx1_3)
                                 - jnp.maximum(px0 * 3.0, tx0_3), 0.0)
                h = jnp.maximum(jnp.minimum(py1, ty1)
                                - jnp.maximum(py0, ty0), 0.0)
                lm = plT_ref[c, 0:1, s] == tl
                valid = jnp.logical_and(w3 * h >= apc + atc, lm)
                r = jnp.any(valid, axis=0, keepdims=True)
                rows[g] = r if rows[g] is None else jnp.logical_or(rows[g], r)
        acts.append(jnp.concatenate([r.astype(jnp.float32) for r in rows],
                                    axis=0))

    kc_vec = [jnp.sum(a, keepdims=True).astype(jnp.int32).reshape(1, 1)
              for a in acts]
    kc = [jnp.sum(a).astype(jnp.int32) for a in acts]
    kmax = jnp.maximum(kc[0], jnp.maximum(kc[1], kc[2]))

    thr = thr_ref[...]
    rec = rec_ref[...]
    lane = jax.lax.broadcasted_iota(jnp.int32, (1, 1, _MPAD), 2)
    tx0s = tbt_ref[:, 0:1, :]
    ty0s = tbt_ref[:, 1:2, :]
    tx1s = tbt_ref[:, 2:3, :]
    ty1s = tbt_ref[:, 3:4, :]
    area_t = (tx1s - tx0s) * (ty1s - ty0s)
    tls = tlabf_ref[...]

    def step(k, carry):
        new_carry = []
        cols = {q: [] for q in range(6)}
        ranks = []
        valids = []
        for c in range(3):
            act = carry[c] > 0.5
            svc = sv_ref[c]
            sel = jnp.where(act, svc, -1.0)
            smax = jnp.max(sel)
            cand = jnp.logical_and(sel == smax, act)
            p_raw = jnp.min(jnp.where(cand, flat, _BIG))
            p = jnp.minimum(p_raw, _N - 1)
            new_carry.append(jnp.where(flat == p_raw, 0.0, carry[c]))
            row = pb_ref[c, pl.ds(p, 1), :]
            for q in range(4):
                cols[q].append(row[0:1, q:q + 1])
            cols[4].append(labf_ref[c, pl.ds(p, 1), 0:1])
            s_c = jnp.max(sel, axis=(0, 1), keepdims=True)
            gt = jnp.sum((svc > s_c).astype(jnp.int32), keepdims=True
                         ).reshape(1, 1)
            tie = jnp.sum(jnp.logical_and(svc == s_c, flat < p
                                          ).astype(jnp.int32), keepdims=True
                          ).reshape(1, 1)
            ranks.append(gt + tie)
            valids.append((jnp.full((1, 1), k, jnp.int32) < kc_vec[c]
                           ).astype(jnp.float32))
        px0 = jnp.concatenate(cols[0], axis=0).reshape(3, 1, 1)
        py0 = jnp.concatenate(cols[1], axis=0).reshape(3, 1, 1)
        px1 = jnp.concatenate(cols[2], axis=0).reshape(3, 1, 1)
        py1 = jnp.concatenate(cols[3], axis=0).reshape(3, 1, 1)
        lab = jnp.concatenate(cols[4], axis=0).reshape(3, 1, 1)
        rank3 = jnp.concatenate(ranks, axis=0).reshape(3, 1, 1)
        valid3 = jnp.concatenate(valids, axis=0).reshape(3, 1, 1) > 0.5

        area_p = (px1 - px0) * (py1 - py0)
        w = jnp.clip(jnp.minimum(px1, tx1s) - jnp.maximum(px0, tx0s), 0.0)
        h = jnp.clip(jnp.minimum(py1, ty1s) - jnp.maximum(py0, ty0s), 0.0)
        inter = w * h
        union = area_p + area_t - inter
        iou = inter / jnp.maximum(union, 1e-9)
        iou_m = jnp.where(lab == tls, iou, -1.0)
        iou_m = jnp.where(valid3, iou_m, -1.0)

        matched = matched_s[...]
        avail = jnp.logical_and(iou_m >= thr, matched < 0.5)
        masked = jnp.where(avail, jnp.broadcast_to(iou_m, matched.shape), -1.0)
        bestv = jnp.max(masked, axis=2, keepdims=True)
        has = bestv > 0.0
        idxs = jnp.where(masked == bestv,
                         jnp.broadcast_to(lane, masked.shape),
                         jnp.int32(_MPAD * 2))
        besti = jnp.min(idxs, axis=2, keepdims=True)
        newm = jnp.logical_and(lane == besti, has)
        matched_s[...] = jnp.where(newm, 1.0, matched)
        ctp = ctp_s[...] + has.astype(jnp.float32)
        ctp_s[...] = ctp
        denom = jnp.where(valid3, (rank3 + 1).astype(jnp.float32), 4096.0)
        prec = ctp / denom
        recall = ctp / float(_M)
        acc_s[...] = jnp.maximum(acc_s[...],
                                 jnp.where(recall >= rec, prec, 0.0))
        return tuple(new_carry)

    jax.lax.fori_loop(0, kmax, step, (acts[0], acts[1], acts[2]))

    ap = jnp.sum(acc_s[:, 0:10, 0:101], axis=(1, 2)) / float(10 * 101)
    out_ref[...] = jnp.broadcast_to(ap[:, None], (3, 128))


@jax.jit
def kernel(pred_boxes, pred_scores, pred_labels, tgt_boxes, tgt_labels):
    f32 = jnp.float32
    labf = pred_labels.astype(f32)[..., None]
    pbT = jnp.pad(jnp.transpose(pred_boxes, (0, 2, 1)),
                  ((0, 0), (0, 0), (0, _NPAD - _N)),
                  constant_values=-1e6)
    plT = jnp.pad(pred_labels.astype(f32)[:, None, :],
                  ((0, 0), (0, 0), (0, _NPAD - _N)),
                  constant_values=-2.0)
    tbt = jnp.pad(jnp.transpose(tgt_boxes, (0, 2, 1)),
                  ((0, 0), (0, 0), (0, _MPAD - _M)))
    tlabf = jnp.pad(tgt_labels, ((0, 0), (0, _MPAD - _M)),
                    constant_values=-1).astype(f32)[:, None, :]
    sv = jnp.pad(pred_scores, ((0, 0), (0, _NPAD - _N)),
                 constant_values=-1.0).reshape(3, _TPAD, _RPAD)
    thr = jnp.pad(jnp.arange(0.5, 0.999, 0.05, dtype=f32),
                  (0, _TPAD - 10), constant_values=2.0).reshape(1, _TPAD, 1)
    rec = jnp.pad(jnp.linspace(0.0, 1.0, 101, dtype=f32),
                  (0, _RPAD - 101), constant_values=2.0).reshape(1, 1, _RPAD)

    full = lambda shape: pl.BlockSpec(shape, lambda: (0,) * len(shape))
    out = pl.pallas_call(
        _body,
        in_specs=[
            full((3, _N, 4)),
            full((3, _N, 1)),
            full((3, 4, _NPAD)),
            full((3, 1, _NPAD)),
            full((3, 4, _MPAD)),
            full((3, 1, _MPAD)),
            full((3, _TPAD, _RPAD)),
            full((1, _TPAD, 1)),
            full((1, 1, _RPAD)),
        ],
        out_specs=pl.BlockSpec((3, 128), lambda: (0, 0)),
        out_shape=jax.ShapeDtypeStruct((3, 128), jnp.float32),
        scratch_shapes=[
            pltpu.VMEM((3, _TPAD, _MPAD), jnp.float32),
            pltpu.VMEM((3, _TPAD, 1), jnp.float32),
            pltpu.VMEM((3, _TPAD, _RPAD), jnp.float32),
        ],
    )(pred_boxes, labf, pbT, plT, tbt, tlabf, sv, thr, rec)
    return out[:, 0]
